# baseline (device time: 18854 ns/iter reference)
import jax
import jax.numpy as jnp
from jax import lax
from jax.experimental import pallas as pl
from jax.experimental.pallas import tpu as pltpu

N_DEV = 32
G = 4
S = 8
BLK = 32


def kernel(x, w_mat):
    m_total, k_local = x.shape
    k_total, n = w_mat.shape

    def body(
        x_hbm, w_hbm, out_ref,
        x_ref,
        w_ref,
        s1_ref,
        r1_ref,
        r2s_ref,
        r2_ref,
        x_sem, w_sem,
        s1_send, s1_recv,
        s2_send, s2_recv,
        s2_ready,
    ):
        me = lax.axis_index("i")
        g = me // S
        lam = me % S

        xcp = pltpu.make_async_copy(x_hbm, x_ref, x_sem)
        xcp.start()
        wcp = pltpu.make_async_copy(w_hbm, w_ref, w_sem)
        wcp.start()

        barrier = pltpu.get_barrier_semaphore()
        for d in range(1, S):
            peer = g * S + (lam + d) % S
            pl.semaphore_signal(
                barrier, inc=1,
                device_id=(peer,), device_id_type=pl.DeviceIdType.MESH,
            )
        for e in range(1, G):
            peer = ((g + e) % G) * S + lam
            pl.semaphore_signal(
                s2_ready, inc=1,
                device_id=(peer,), device_id_type=pl.DeviceIdType.MESH,
            )

        xcp.wait()
        for p in range(N_DEV):
            gp, lp = p // S, p % S
            s1_ref[lp, gp, :, :] = (
                x_ref[pl.ds(p * BLK, BLK), :].T.astype(jnp.bfloat16)
            )

        pl.semaphore_wait(barrier, S - 1)

        s1_rdmas = []
        for d in range(1, S):
            lp = (lam + d) % S
            rdma = pltpu.make_async_remote_copy(
                src_ref=s1_ref.at[lp],
                dst_ref=r1_ref.at[lam],
                send_sem=s1_send.at[d],
                recv_sem=s1_recv.at[d],
                device_id=(g * S + lp,),
                device_id_type=pl.DeviceIdType.MESH,
            )
            rdma.start()
            s1_rdmas.append(rdma)
        r1_ref[lam] = s1_ref[lam]
        for r in s1_rdmas:
            r.wait_recv()
        for l in range(S):
            r2s_ref[:, l] = r1_ref[l]

        pl.semaphore_wait(s2_ready, G - 1)
        s2_rdmas = []
        for e in range(1, G):
            gp = (g + e) % G
            rdma = pltpu.make_async_remote_copy(
                src_ref=r2s_ref.at[gp],
                dst_ref=r2_ref.at[g],
                send_sem=s2_send.at[e],
                recv_sem=s2_recv.at[e],
                device_id=(gp * S + lam,),
                device_id_type=pl.DeviceIdType.MESH,
            )
            rdma.start()
            s2_rdmas.append(rdma)
        r2_ref[g] = r2s_ref[g]

        wcp.wait()

        def bundle_dot(q):
            xq = jnp.reshape(r2_ref[q], (S * BLK, BLK)).astype(jnp.float32)
            return lax.dot_general(
                xq,
                w_ref[pl.ds(q * S * BLK, S * BLK), :],
                dimension_numbers=(((0,), (0,)), ((), ())),
                preferred_element_type=jnp.float32,
            )

        acc = bundle_dot(g)
        for e in range(1, G):
            s2_rdmas[e - 1].wait_recv()
            acc = acc + bundle_dot((g - e) % G)
        out_ref[:, :] = acc

        for r in s1_rdmas:
            r.wait_send()
        for r in s2_rdmas:
            r.wait_send()

    return pl.pallas_call(
        body,
        out_shape=jax.ShapeDtypeStruct((BLK, n), jnp.float32),
        in_specs=[
            pl.BlockSpec(memory_space=pl.ANY),
            pl.BlockSpec(memory_space=pl.ANY),
        ],
        out_specs=pl.BlockSpec(memory_space=pltpu.VMEM),
        scratch_shapes=[
            pltpu.VMEM((m_total, k_local), x.dtype),
            pltpu.VMEM((k_total, n), w_mat.dtype),
            pltpu.VMEM((S, G, BLK, BLK), jnp.bfloat16),
            pltpu.VMEM((S, G, BLK, BLK), jnp.bfloat16),
            pltpu.VMEM((G, S, BLK, BLK), jnp.bfloat16),
            pltpu.VMEM((G, S, BLK, BLK), jnp.bfloat16),
            pltpu.SemaphoreType.DMA,
            pltpu.SemaphoreType.DMA,
            pltpu.SemaphoreType.DMA((S,)),
            pltpu.SemaphoreType.DMA((S,)),
            pltpu.SemaphoreType.DMA((G,)),
            pltpu.SemaphoreType.DMA((G,)),
            pltpu.SemaphoreType.REGULAR,
        ],
        compiler_params=pltpu.CompilerParams(collective_id=0),
    )(x, w_mat)


# device time: 18549 ns/iter; 1.0164x vs baseline; 1.0164x over previous
import jax
import jax.numpy as jnp
from jax import lax
from jax.experimental import pallas as pl
from jax.experimental.pallas import tpu as pltpu

N_DEV = 32
G = 8
S = 4
BLK = 32


def kernel(x, w_mat):
    m_total, k_local = x.shape
    k_total, n = w_mat.shape

    def body(
        x_hbm, w_hbm, out_ref,
        x_ref,
        w_ref,
        s1_ref,
        r1_ref,
        r2_ref,
        x_sem, w_sem,
        s1_send, s1_recv,
        s2_send, s2_recv,
        s2_ready,
    ):
        me = lax.axis_index("i")
        g = me // S
        lam = me % S

        xcp = pltpu.make_async_copy(x_hbm, x_ref, x_sem)
        xcp.start()
        wcp = pltpu.make_async_copy(w_hbm, w_ref, w_sem)
        wcp.start()

        barrier = pltpu.get_barrier_semaphore()
        for d in range(1, S):
            peer = g * S + (lam + d) % S
            pl.semaphore_signal(
                barrier, inc=1,
                device_id=(peer,), device_id_type=pl.DeviceIdType.MESH,
            )
        for e in range(1, G):
            peer = ((g + e) % G) * S + lam
            pl.semaphore_signal(
                s2_ready, inc=1,
                device_id=(peer,), device_id_type=pl.DeviceIdType.MESH,
            )

        xcp.wait()
        for p in range(N_DEV):
            gp, lp = p // S, p % S
            s1_ref[lp, gp, :, :] = (
                x_ref[pl.ds(p * BLK, BLK), :].T.astype(jnp.bfloat16)
            )

        pl.semaphore_wait(barrier, S - 1)

        s1_rdmas = []
        for d in range(1, S):
            lp = (lam + d) % S
            rdma = pltpu.make_async_remote_copy(
                src_ref=s1_ref.at[lp],
                dst_ref=r1_ref.at[:, lam],
                send_sem=s1_send.at[d],
                recv_sem=s1_recv.at[d],
                device_id=(g * S + lp,),
                device_id_type=pl.DeviceIdType.MESH,
            )
            rdma.start()
            s1_rdmas.append(rdma)
        r1_ref[:, lam] = s1_ref[lam]
        for r in s1_rdmas:
            r.wait_recv()

        pl.semaphore_wait(s2_ready, G - 1)
        s2_rdmas = []
        for e in range(1, G):
            gp = (g + e) % G
            rdma = pltpu.make_async_remote_copy(
                src_ref=r1_ref.at[gp],
                dst_ref=r2_ref.at[g],
                send_sem=s2_send.at[e],
                recv_sem=s2_recv.at[e],
                device_id=(gp * S + lam,),
                device_id_type=pl.DeviceIdType.MESH,
            )
            rdma.start()
            s2_rdmas.append(rdma)
        r2_ref[g] = r1_ref[g]

        wcp.wait()

        def bundle_dot(q):
            xq = jnp.reshape(r2_ref[q], (S * BLK, BLK)).astype(jnp.float32)
            return lax.dot_general(
                xq,
                w_ref[pl.ds(q * S * BLK, S * BLK), :],
                dimension_numbers=(((0,), (0,)), ((), ())),
                preferred_element_type=jnp.float32,
            )

        acc = bundle_dot(g)
        for e in range(1, G):
            s2_rdmas[e - 1].wait_recv()
            acc = acc + bundle_dot((g - e) % G)
        out_ref[:, :] = acc

        for r in s1_rdmas:
            r.wait_send()
        for r in s2_rdmas:
            r.wait_send()

    return pl.pallas_call(
        body,
        out_shape=jax.ShapeDtypeStruct((BLK, n), jnp.float32),
        in_specs=[
            pl.BlockSpec(memory_space=pl.ANY),
            pl.BlockSpec(memory_space=pl.ANY),
        ],
        out_specs=pl.BlockSpec(memory_space=pltpu.VMEM),
        scratch_shapes=[
            pltpu.VMEM((m_total, k_local), x.dtype),
            pltpu.VMEM((k_total, n), w_mat.dtype),
            pltpu.VMEM((S, G, BLK, BLK), jnp.bfloat16),
            pltpu.VMEM((G, S, BLK, BLK), jnp.bfloat16),
            pltpu.VMEM((G, S, BLK, BLK), jnp.bfloat16),
            pltpu.SemaphoreType.DMA,
            pltpu.SemaphoreType.DMA,
            pltpu.SemaphoreType.DMA((S,)),
            pltpu.SemaphoreType.DMA((S,)),
            pltpu.SemaphoreType.DMA((G,)),
            pltpu.SemaphoreType.DMA((G,)),
            pltpu.SemaphoreType.REGULAR,
        ],
        compiler_params=pltpu.CompilerParams(collective_id=0),
    )(x, w_mat)
